# Initial kernel scaffold; baseline (speedup 1.0000x reference)
import jax
import jax.numpy as jnp
from jax.experimental import pallas as pl


def kernel(x, edge_index, edge_type, node2graph, Wrel0, brel0, Wself0, bself0, Wrel1, brel1, Wself1, bself1, Wrel2, brel2, Wself2, bself2):
    raise NotImplementedError("write your pallas kernel here")


# speedup vs baseline: 6.5296x; 6.5296x over previous
import jax
import jax.numpy as jnp
from jax import lax
from jax.experimental import pallas as pl
from jax.experimental.pallas import tpu as pltpu
from jax.experimental.pallas import tpu_sc as plsc

NN = 10000
EE = 320000
DD = 128
RR = 4
GG = 8
LL = 3

NC = 2
NS = 16
NW = NC * NS

CH = 128
NCHUNK = EE // CH
CPW = -(-NCHUNK // NW)
NP = 10240
RPT = NP // NS

BN = 2000
NB = NN // BN


def _edge_agg_body(t_hbm, gidx_hbm, dst_hbm, out_hbm, gbuf, dbuf, rows, acc, sem):
    c = lax.axis_index("c")
    s = lax.axis_index("s")
    w = s * NC + c

    zero16 = jnp.zeros((16,), jnp.float32)

    def zrow(i, _):
        for j in range(DD // 16):
            rows[i, pl.ds(j * 16, 16)] = zero16
        return 0

    lax.fori_loop(0, CH, zrow, 0)
    for k in range(RPT // CH):
        pltpu.sync_copy(rows, acc.at[pl.ds(s * RPT + k * CH, CH)])
    plsc.subcore_barrier()

    def body(i, _):
        cid = w + NW * i

        @pl.when(cid < NCHUNK)
        def _():
            pltpu.sync_copy(gidx_hbm.at[cid], gbuf)
            pltpu.sync_copy(dst_hbm.at[cid], dbuf)
            pltpu.async_copy(t_hbm.at[gbuf], rows, sem).wait()
            pltpu.sync_copy(rows, acc.at[dbuf], add=True)

        return 0

    lax.fori_loop(0, CPW, body, 0)

    plsc.subcore_barrier()
    pltpu.sync_copy(acc.at[pl.ds(s * RPT, RPT)],
                    out_hbm.at[c, pl.ds(s * RPT, RPT)])


_edge_agg = pl.kernel(
    _edge_agg_body,
    out_type=jax.ShapeDtypeStruct((NC, NP, DD), jnp.float32),
    mesh=plsc.VectorSubcoreMesh(core_axis_name="c", subcore_axis_name="s",
                                num_cores=NC, num_subcores=NS),
    scratch_types=[
        pltpu.VMEM((CH,), jnp.int32),
        pltpu.VMEM((CH,), jnp.int32),
        pltpu.VMEM((CH, DD), jnp.float32),
        pltpu.VMEM_SHARED((NP, DD), jnp.float32),
        pltpu.SemaphoreType.DMA,
    ],
)


def _xform(h, wr_ref, ws_ref, b_ref, t_ref, s_ref):
    for r in range(RR):
        t_ref[r] = jnp.dot(h, wr_ref[r * DD:(r + 1) * DD, :],
                           preferred_element_type=jnp.float32)
    s_ref[...] = jnp.dot(h, ws_ref[...],
                         preferred_element_type=jnp.float32) + b_ref[...]


def _l0_body(x_ref, wr_ref, ws_ref, b_ref, t_ref, s_ref):
    _xform(x_ref[...], wr_ref, ws_ref, b_ref, t_ref, s_ref)


def _li_body(p_ref, sp_ref, wr_ref, ws_ref, b_ref, t_ref, s_ref):
    h = jnp.maximum(p_ref[0] + p_ref[1] + sp_ref[...], 0.0)
    _xform(h, wr_ref, ws_ref, b_ref, t_ref, s_ref)


def _readout_body(n2g_ref, p_ref, sp_ref, out_ref):
    h = jnp.maximum(p_ref[0] + p_ref[1] + sp_ref[...], 0.0)
    n2g = n2g_ref[0, 0, :]
    onehot = (n2g[:, None] == lax.broadcasted_iota(jnp.int32, (BN, GG), 1)
              ).astype(jnp.float32)
    contrib = lax.dot_general(onehot, h, (((0,), (0,)), ((), ())),
                              preferred_element_type=jnp.float32)

    @pl.when(pl.program_id(0) == 0)
    def _():
        out_ref[...] = jnp.zeros_like(out_ref)

    out_ref[...] += contrib


_W_SPECS = [
    pl.BlockSpec((RR * DD, DD), lambda i: (0, 0)),
    pl.BlockSpec((DD, DD), lambda i: (0, 0)),
    pl.BlockSpec((1, DD), lambda i: (0, 0)),
]
_TS_OUT = dict(
    out_specs=[
        pl.BlockSpec((RR, BN, DD), lambda i: (0, i, 0)),
        pl.BlockSpec((BN, DD), lambda i: (i, 0)),
    ],
    out_shape=[
        jax.ShapeDtypeStruct((RR, NN, DD), jnp.float32),
        jax.ShapeDtypeStruct((NN, DD), jnp.float32),
    ],
)

_l0 = pl.pallas_call(
    _l0_body,
    grid=(NB,),
    in_specs=[pl.BlockSpec((BN, DD), lambda i: (i, 0))] + _W_SPECS,
    **_TS_OUT,
)

_li = pl.pallas_call(
    _li_body,
    grid=(NB,),
    in_specs=[
        pl.BlockSpec((NC, BN, DD), lambda i: (0, i, 0)),
        pl.BlockSpec((BN, DD), lambda i: (i, 0)),
    ] + _W_SPECS,
    **_TS_OUT,
)

_readout = pl.pallas_call(
    _readout_body,
    grid=(NB,),
    in_specs=[
        pl.BlockSpec((1, 1, BN), lambda i: (i, 0, 0)),
        pl.BlockSpec((NC, BN, DD), lambda i: (0, i, 0)),
        pl.BlockSpec((BN, DD), lambda i: (i, 0)),
    ],
    out_specs=pl.BlockSpec((GG, DD), lambda i: (0, 0)),
    out_shape=jax.ShapeDtypeStruct((GG, DD), jnp.float32),
)


def kernel(x, edge_index, edge_type, node2graph,
           Wrel0, brel0, Wself0, bself0,
           Wrel1, brel1, Wself1, bself1,
           Wrel2, brel2, Wself2, bself2):
    src = edge_index[0].astype(jnp.int32)
    dst = edge_index[1].astype(jnp.int32)
    gidx = (edge_type.astype(jnp.int32) * NN + src).reshape(NCHUNK, CH)
    dst2d = dst.reshape(NCHUNK, CH)
    n2g3d = node2graph.astype(jnp.int32).reshape(NB, 1, BN)

    Wrels = (Wrel0, Wrel1, Wrel2)
    Wselfs = (Wself0, Wself1, Wself2)
    biases = tuple((br + bs).reshape(1, DD)
                   for br, bs in ((brel0, bself0), (brel1, bself1),
                                  (brel2, bself2)))

    T, S = _l0(x, Wrels[0], Wselfs[0], biases[0])
    for i in range(1, LL):
        P = _edge_agg(T.reshape(RR * NN, DD), gidx, dst2d)
        T, S = _li(P, S, Wrels[i], Wselfs[i], biases[i])
    P = _edge_agg(T.reshape(RR * NN, DD), gidx, dst2d)
    return _readout(n2g3d, P, S)


# speedup vs baseline: 11.8771x; 1.8190x over previous
import jax
import jax.numpy as jnp
from jax import lax
from jax.experimental import pallas as pl
from jax.experimental.pallas import tpu as pltpu
from jax.experimental.pallas import tpu_sc as plsc

NN = 10000
EE = 320000
DD = 128
RR = 4
GG = 8
LL = 3

NC = 2
NS = 16
NW = NC * NS

CH = 128
NCHUNK = EE // CH
CPW = -(-NCHUNK // NW)
NP = 10240
RPT = NP // NS

BN = 2000
NB = NN // BN


def _edge_agg_body(t_hbm, idx_hbm, out_hbm,
                   ibufs, rowbufs, acc, semi, semg):
    c = lax.axis_index("c")
    s = lax.axis_index("s")
    w = s * NC + c

    zero16 = jnp.zeros((16,), jnp.float32)

    def zrow(i, _):
        for j in range(DD // 16):
            rowbufs[0][i, pl.ds(j * 16, 16)] = zero16
        return 0

    lax.fori_loop(0, CH, zrow, 0)
    for k in range(RPT // CH):
        pltpu.sync_copy(rowbufs[0], acc.at[pl.ds(s * RPT + k * CH, CH)])
    plsc.subcore_barrier()

    def idx_start(i, b):
        pltpu.async_copy(idx_hbm.at[w + NW * i], ibufs[b], semi[b])

    def idx_wait(i, b):
        pltpu.make_async_copy(idx_hbm.at[w + NW * i], ibufs[b], semi[b]).wait()

    def gather_start(b):
        pltpu.async_copy(t_hbm.at[ibufs[b].at[0]], rowbufs[b], semg[b])

    def gather_wait(b):
        pltpu.make_async_copy(t_hbm.at[ibufs[b].at[0]], rowbufs[b],
                              semg[b]).wait()

    def valid(i):
        return w + NW * i < NCHUNK

    idx_start(0, 0)
    idx_wait(0, 0)
    gather_start(0)
    idx_start(1, 1)

    def body(j, _):
        for u in range(2):
            i = 2 * j + u
            b, nb = u, 1 - u

            @pl.when(valid(i + 1))
            def _():
                idx_wait(i + 1, nb)
                gather_start(nb)

            @pl.when(valid(i))
            def _():
                gather_wait(b)
                pltpu.sync_copy(rowbufs[b], acc.at[ibufs[b].at[1]], add=True)

            @pl.when(valid(i + 2))
            def _():
                idx_start(i + 2, b)

        return 0

    lax.fori_loop(0, (CPW + 1) // 2, body, 0)

    plsc.subcore_barrier()
    pltpu.sync_copy(acc.at[pl.ds(s * RPT, RPT)],
                    out_hbm.at[c, pl.ds(s * RPT, RPT)])


_edge_agg = pl.kernel(
    _edge_agg_body,
    out_type=jax.ShapeDtypeStruct((NC, NP, DD), jnp.float32),
    mesh=plsc.VectorSubcoreMesh(core_axis_name="c", subcore_axis_name="s",
                                num_cores=NC, num_subcores=NS),
    scratch_types=[
        [pltpu.VMEM((2, CH), jnp.int32)] * 2,
        [pltpu.VMEM((CH, DD), jnp.float32)] * 2,
        pltpu.VMEM_SHARED((NP, DD), jnp.float32),
        [pltpu.SemaphoreType.DMA] * 2,
        [pltpu.SemaphoreType.DMA] * 2,
    ],
)


def _xform(h, wr_ref, ws_ref, b_ref, t_ref, s_ref):
    for r in range(RR):
        t_ref[r] = jnp.dot(h, wr_ref[r * DD:(r + 1) * DD, :],
                           preferred_element_type=jnp.float32)
    s_ref[...] = jnp.dot(h, ws_ref[...],
                         preferred_element_type=jnp.float32) + b_ref[...]


def _l0_body(x_ref, wr_ref, ws_ref, b_ref, t_ref, s_ref):
    _xform(x_ref[...], wr_ref, ws_ref, b_ref, t_ref, s_ref)


def _li_body(p_ref, sp_ref, wr_ref, ws_ref, b_ref, t_ref, s_ref):
    h = jnp.maximum(p_ref[0] + p_ref[1] + sp_ref[...], 0.0)
    _xform(h, wr_ref, ws_ref, b_ref, t_ref, s_ref)


def _readout_body(n2g_ref, p_ref, sp_ref, out_ref):
    h = jnp.maximum(p_ref[0] + p_ref[1] + sp_ref[...], 0.0)
    n2g = n2g_ref[0, 0, :]
    onehot = (n2g[:, None] == lax.broadcasted_iota(jnp.int32, (BN, GG), 1)
              ).astype(jnp.float32)
    contrib = lax.dot_general(onehot, h, (((0,), (0,)), ((), ())),
                              preferred_element_type=jnp.float32)

    @pl.when(pl.program_id(0) == 0)
    def _():
        out_ref[...] = jnp.zeros_like(out_ref)

    out_ref[...] += contrib


_W_SPECS = [
    pl.BlockSpec((RR * DD, DD), lambda i: (0, 0)),
    pl.BlockSpec((DD, DD), lambda i: (0, 0)),
    pl.BlockSpec((1, DD), lambda i: (0, 0)),
]
_TS_OUT = dict(
    out_specs=[
        pl.BlockSpec((RR, BN, DD), lambda i: (0, i, 0)),
        pl.BlockSpec((BN, DD), lambda i: (i, 0)),
    ],
    out_shape=[
        jax.ShapeDtypeStruct((RR, NN, DD), jnp.float32),
        jax.ShapeDtypeStruct((NN, DD), jnp.float32),
    ],
)

_l0 = pl.pallas_call(
    _l0_body,
    grid=(NB,),
    in_specs=[pl.BlockSpec((BN, DD), lambda i: (i, 0))] + _W_SPECS,
    **_TS_OUT,
)

_li = pl.pallas_call(
    _li_body,
    grid=(NB,),
    in_specs=[
        pl.BlockSpec((NC, BN, DD), lambda i: (0, i, 0)),
        pl.BlockSpec((BN, DD), lambda i: (i, 0)),
    ] + _W_SPECS,
    **_TS_OUT,
)

_readout = pl.pallas_call(
    _readout_body,
    grid=(NB,),
    in_specs=[
        pl.BlockSpec((1, 1, BN), lambda i: (i, 0, 0)),
        pl.BlockSpec((NC, BN, DD), lambda i: (0, i, 0)),
        pl.BlockSpec((BN, DD), lambda i: (i, 0)),
    ],
    out_specs=pl.BlockSpec((GG, DD), lambda i: (0, 0)),
    out_shape=jax.ShapeDtypeStruct((GG, DD), jnp.float32),
)


def kernel(x, edge_index, edge_type, node2graph,
           Wrel0, brel0, Wself0, bself0,
           Wrel1, brel1, Wself1, bself1,
           Wrel2, brel2, Wself2, bself2):
    src = edge_index[0].astype(jnp.int32)
    dst = edge_index[1].astype(jnp.int32)
    gidx = (edge_type.astype(jnp.int32) * NN + src).reshape(NCHUNK, CH)
    idx2 = jnp.stack([gidx, dst.reshape(NCHUNK, CH)], axis=1)
    n2g3d = node2graph.astype(jnp.int32).reshape(NB, 1, BN)

    Wrels = (Wrel0, Wrel1, Wrel2)
    Wselfs = (Wself0, Wself1, Wself2)
    biases = tuple((br + bs).reshape(1, DD)
                   for br, bs in ((brel0, bself0), (brel1, bself1),
                                  (brel2, bself2)))

    T, S = _l0(x, Wrels[0], Wselfs[0], biases[0])
    for i in range(1, LL):
        P = _edge_agg(T.reshape(RR * NN, DD), idx2)
        T, S = _li(P, S, Wrels[i], Wselfs[i], biases[i])
    P = _edge_agg(T.reshape(RR * NN, DD), idx2)
    return _readout(n2g3d, P, S)
